# baseline (speedup 1.0000x reference)
import functools

import jax
import jax.numpy as jnp
from jax import lax
from jax.experimental import pallas as pl
from jax.experimental.pallas import tpu as pltpu
from jax.experimental.pallas import tpu_sc as plsc

N = 10000
M = 10000
NNZ = 160000
D = 256
DH = 128
NC = 2
NT = 16
K = 128
PT = 10240
CH = PT // K
PB = 20
NNZ_PAD = NT * PT
SACC = 10240
ZR = 32
ROWS = 1000
HPT = NNZ // NT


def _mesh():
    return plsc.VectorSubcoreMesh(core_axis_name="c", subcore_axis_name="s")


@functools.cache
def _build_sc_degrees():
    @functools.partial(
        pl.kernel,
        out_type=(
            jax.ShapeDtypeStruct((SACC,), jnp.float32),
            jax.ShapeDtypeStruct((SACC,), jnp.float32),
        ),
        mesh=_mesh(),
        compiler_params=pltpu.CompilerParams(needs_layout_passes=False),
        scratch_types=[
            pltpu.VMEM((HPT,), jnp.int32),
            pltpu.VMEM((SACC // 16, 16), jnp.float32),
            pltpu.VMEM((5, 128), jnp.int32),
            pltpu.VMEM((SACC // NT // 16, 16), jnp.float32),
            pltpu.VMEM((SACC // NT,), jnp.float32),
            pltpu.VMEM_SHARED((SACC // 16, 16), jnp.float32),
        ],
    )
    def sc_degrees(node_hbm, edge_hbm, apad_hbm, epad_hbm,
                   idx_v, hist_v, riota, cbuf, obuf, acc_sh):
        c = lax.axis_index("c")
        t = lax.axis_index("s")
        nrow = SACC // 16

        def riota_set(i, carry):
            kk = i // 8
            m = i % 8
            riota[kk, pl.ds(m * 16, 16)] = (
                lax.iota(jnp.int32, 16) + jnp.full((16,), i * 16, jnp.int32)
            )
            return carry

        lax.fori_loop(0, nrow // 16, riota_set, 0)

        def pipeline(src_hbm, out_hbm, is_rsqrt):
            def zero(i, carry):
                hist_v[i // 8, pl.ds((i % 8) * 16, 16)] = jnp.zeros(
                    (16,), jnp.float32)
                return carry

            lax.fori_loop(0, nrow * 16 // 16, zero, 0)
            pltpu.sync_copy(src_hbm.at[pl.ds(t * HPT, HPT)], idx_v)

            ones = jnp.ones((16,), jnp.float32)

            def count(i, carry):
                idx = idx_v[pl.ds(i * 16, 16)]
                plsc.addupdate_scatter(
                    hist_v, [lax.shift_right_logical(idx, 4), idx & 15], ones)
                return carry

            lax.fori_loop(0, HPT // 16, count, 0)
            plsc.subcore_barrier()

            @pl.when(t == 0)
            def _():
                pltpu.sync_copy(hist_v, acc_sh)

            plsc.subcore_barrier()

            @pl.when(t != 0)
            def _():
                for k in range(5):
                    pltpu.sync_copy(
                        hist_v.at[pl.ds(k * 128, 128)],
                        acc_sh.at[riota.at[k]],
                        add=True,
                    )

            plsc.subcore_barrier()

            rpt16 = SACC // NT // 16
            pltpu.sync_copy(acc_sh.at[pl.ds(t * rpt16, rpt16)], cbuf)

            def conv(i, carry):
                d = jnp.maximum(cbuf[i, :], 1.0)
                if is_rsqrt:
                    bits = plsc.bitcast(d, jnp.int32)
                    bits = jnp.full((16,), 0x5F3759DF, jnp.int32) - (
                        lax.shift_right_logical(bits, 1))
                    y = plsc.bitcast(bits, jnp.float32)
                    for _ in range(3):
                        y = y * (1.5 - 0.5 * d * y * y)
                    obuf[pl.ds(i * 16, 16)] = y
                else:
                    obuf[pl.ds(i * 16, 16)] = 1.0 / d
                return carry

            lax.fori_loop(0, rpt16, conv, 0)
            pltpu.sync_copy(obuf, out_hbm.at[pl.ds(t * (SACC // NT), SACC // NT)])

        @pl.when(c == 0)
        def _():
            pipeline(node_hbm, apad_hbm, True)

        @pl.when(c == 1)
        def _():
            pipeline(edge_hbm, epad_hbm, False)

    return sc_degrees


def _sc_degrees_call(node_idx, edge_idx):
    return _build_sc_degrees()(node_idx, edge_idx)


@functools.cache
def _build_sc_lmm():
    @functools.partial(
        pl.kernel,
        out_type=(
            jax.ShapeDtypeStruct((NC, SACC, DH), jnp.float32),
            jax.ShapeDtypeStruct((NC, SACC, DH), jnp.float32),
        ),
        mesh=_mesh(),
        compiler_params=pltpu.CompilerParams(needs_layout_passes=False),
        scratch_types=[
            pltpu.VMEM((2, PB, 2, K), jnp.int32),
            pltpu.VMEM((2, K, DH), jnp.float32),
            pltpu.VMEM((SACC // NT,), jnp.float32),
            pltpu.VMEM((ZR, DH), jnp.float32),
            pltpu.VMEM_SHARED((SACC, DH), jnp.float32),
            pltpu.SemaphoreType.DMA,
            pltpu.SemaphoreType.DMA,
        ],
    )
    def sc_lmm(table_hbm, pairsa_hbm, pairsb_hbm, scale_hbm,
               mid_hbm, out_hbm, pbuf, rows, ebuf, zbuf, acc, gsem0, gsem1):
        c = lax.axis_index("c")
        t = lax.axis_index("s")
        gsems = (gsem0, gsem1)
        rpt = SACC // NT
        base = t * rpt
        nblk = rpt // K

        def zset(i, carry):
            r = i // (DH // 16)
            col = (i % (DH // 16)) * 16
            zbuf[r, pl.ds(col, 16)] = jnp.zeros((16,), jnp.float32)
            return carry

        lax.fori_loop(0, ZR * (DH // 16), zset, 0)
        pltpu.sync_copy(scale_hbm.at[pl.ds(base, rpt)], ebuf)

        def zero_acc():
            def zfire(i, carry):
                pltpu.async_copy(zbuf, acc.at[pl.ds(base + i * ZR, ZR)], gsem0)
                return carry

            lax.fori_loop(0, rpt // ZR, zfire, 0)

            def zwait(i, carry):
                pltpu.make_async_copy(
                    zbuf, acc.at[pl.ds(base + i * ZR, ZR)], gsem0
                ).wait()
                return carry

            lax.fori_loop(0, rpt // ZR, zwait, 0)

        def stage(tbl, pairs_hbm, dst_hbm, scaled):
            def load_block(b):
                pltpu.sync_copy(
                    pairs_hbm.at[pl.ds(t * CH + b * PB, PB)],
                    pbuf.at[lax.rem(b, 2)],
                )

            def gidx(j):
                return pbuf.at[lax.rem(j // PB, 2), lax.rem(j, PB), 0]

            def sidx(j):
                return pbuf.at[lax.rem(j // PB, 2), lax.rem(j, PB), 1]

            def fire_gather(slot, j):
                pltpu.async_copy(tbl.at[gidx(j)], rows.at[slot], gsems[slot])

            def wait_gather(slot, j):
                pltpu.make_async_copy(
                    tbl.at[gidx(j)], rows.at[slot], gsems[slot]
                ).wait()

            def scatter_add(slot, j):
                pltpu.sync_copy(rows.at[slot], acc.at[sidx(j)], add=True)

            load_block(0)
            fire_gather(0, 0)

            def body(jj, carry):
                j0 = 2 * jj
                j1 = j0 + 1
                fire_gather(1, j1)
                wait_gather(0, j0)
                scatter_add(0, j0)

                @pl.when(jj != CH // 2 - 1)
                def _():
                    @pl.when(lax.rem(j0 + 2, PB) == 0)
                    def _():
                        load_block((j0 + 2) // PB)

                    fire_gather(0, j0 + 2)

                wait_gather(1, j1)
                scatter_add(1, j1)
                return carry

            lax.fori_loop(0, CH // 2, body, 0)
            plsc.subcore_barrier()

            if not scaled:
                pltpu.sync_copy(
                    acc.at[pl.ds(base, rpt)],
                    dst_hbm.at[c].at[pl.ds(base, rpt)],
                )
            else:
                pltpu.async_copy(acc.at[pl.ds(base, K)], rows.at[0], gsem0)
                dnums = lax.GatherDimensionNumbers(
                    offset_dims=(), collapsed_slice_dims=(0,),
                    start_index_map=(0,))
                for b in range(nblk):
                    sl = b % 2
                    pltpu.make_async_copy(
                        acc.at[pl.ds(base + b * K, K)], rows.at[sl], gsems[sl]
                    ).wait()
                    if b + 1 < nblk:
                        pltpu.async_copy(
                            acc.at[pl.ds(base + (b + 1) * K, K)],
                            rows.at[1 - sl],
                            gsems[1 - sl],
                        )

                    def scale_group(g, carry):
                        e16 = ebuf[pl.ds(b * K + g * 16, 16)]
                        for i in range(16):
                            r = g * 16 + i
                            e = lax.gather(
                                e16,
                                jnp.full((16, 1), i, jnp.int32),
                                dnums,
                                slice_sizes=(1,),
                                mode=lax.GatherScatterMode.PROMISE_IN_BOUNDS,
                            )
                            for q in range(DH // 16):
                                rows[sl, r, pl.ds(q * 16, 16)] = (
                                    rows[sl, r, pl.ds(q * 16, 16)] * e
                                )
                        return carry

                    lax.fori_loop(0, K // 16, scale_group, 0)
                    pltpu.sync_copy(
                        rows.at[sl],
                        dst_hbm.at[c].at[pl.ds(base + b * K, K)],
                    )

        zero_acc()
        plsc.subcore_barrier()
        stage(table_hbm.at[c], pairsa_hbm, mid_hbm, True)
        zero_acc()
        plsc.subcore_barrier()
        stage(mid_hbm.at[c], pairsb_hbm, out_hbm, False)

    return sc_lmm


def _sc_lmm_call(table, pairs_a, pairs_b, scale_pad):
    return _build_sc_lmm()(table, pairs_a, pairs_b, scale_pad)


def _mm1_body(x_ref, w_ref, b_ref, a_ref, out_ref):
    y = lax.dot_general(
        x_ref[...], w_ref[...], (((1,), (1,)), ((), ())),
        preferred_element_type=jnp.float32,
    )
    out_ref[...] = ((y + b_ref[...]) * a_ref[...])[None]


def _tc_mm1(X, W1, b1, acol):
    return pl.pallas_call(
        _mm1_body,
        grid=(NC, N // ROWS),
        in_specs=[
            pl.BlockSpec((ROWS, D), lambda c, i: (i, 0)),
            pl.BlockSpec((DH, D), lambda c, i: (c, 0)),
            pl.BlockSpec((1, DH), lambda c, i: (0, c)),
            pl.BlockSpec((ROWS, 1), lambda c, i: (i, 0)),
        ],
        out_specs=pl.BlockSpec((1, ROWS, DH), lambda c, i: (c, i, 0)),
        out_shape=jax.ShapeDtypeStruct((NC, SACC, DH), jnp.float32),
    )(X, W1, b1.reshape(1, D), acol)


def _mm2_body(z_ref, a_ref, w_ref, b_ref, out_ref):
    k = pl.program_id(2)
    a = a_ref[...]
    h = jnp.maximum(z_ref[0] * a, 0.0)
    p = lax.dot_general(
        h, w_ref[...], (((1,), (1,)), ((), ())),
        preferred_element_type=jnp.float32,
    )

    @pl.when(k == 0)
    def _():
        out_ref[...] = p[None]

    @pl.when(k == 1)
    def _():
        out_ref[...] = ((out_ref[0] + p + b_ref[...]) * a)[None]


def _tc_mm2(Zv, W2, b2, acol):
    return pl.pallas_call(
        _mm2_body,
        grid=(NC, N // ROWS, NC),
        in_specs=[
            pl.BlockSpec((1, ROWS, DH), lambda c, i, k: (k, i, 0)),
            pl.BlockSpec((ROWS, 1), lambda c, i, k: (i, 0)),
            pl.BlockSpec((DH, DH), lambda c, i, k: (c, k)),
            pl.BlockSpec((1, DH), lambda c, i, k: (0, c)),
        ],
        out_specs=pl.BlockSpec((1, ROWS, DH), lambda c, i, k: (c, i, 0)),
        out_shape=jax.ShapeDtypeStruct((NC, SACC, DH), jnp.float32),
    )(Zv, acol, W2, b2.reshape(1, D))


def _final_body(z_ref, a_ref, out_ref):
    out_ref[...] = z_ref[0] * a_ref[...]


def _tc_final(Zv, acol):
    return pl.pallas_call(
        _final_body,
        grid=(NC, N // ROWS),
        in_specs=[
            pl.BlockSpec((1, ROWS, DH), lambda c, i: (c, i, 0)),
            pl.BlockSpec((ROWS, 1), lambda c, i: (i, 0)),
        ],
        out_specs=pl.BlockSpec((ROWS, DH), lambda c, i: (i, c)),
        out_shape=jax.ShapeDtypeStruct((N, D), jnp.float32),
    )(Zv, acol)


def _pack_pairs(gidx, sidx, gmod, spad_base):
    npad = NNZ_PAD - NNZ
    fill = jnp.arange(npad, dtype=jnp.int32)
    g = jnp.concatenate([gidx.astype(jnp.int32), fill % gmod])
    s = jnp.concatenate([sidx.astype(jnp.int32),
                         spad_base + fill % (SACC - spad_base)])
    g3 = g.reshape(NT, CH, 1, K)
    s3 = s.reshape(NT, CH, 1, K)
    return jnp.concatenate([g3, s3], axis=2).reshape(NT * CH, 2, K)


def kernel(X, W1, b1, W2, b2, node_idx, edge_idx):
    pairs_ne = _pack_pairs(node_idx, edge_idx, N, M)
    pairs_en = _pack_pairs(edge_idx, node_idx, M, N)

    apad, einv_pad = _sc_degrees_call(node_idx, edge_idx)
    acol = apad[:N].reshape(N, 1)

    y1 = _tc_mm1(X, W1, b1, acol)
    _, zv = _sc_lmm_call(y1, pairs_ne, pairs_en, einv_pad)
    y2 = _tc_mm2(zv, W2, b2, acol)
    _, zv2 = _sc_lmm_call(y2, pairs_ne, pairs_en, einv_pad)
    return _tc_final(zv2, acol)


# speedup vs baseline: 1.0070x; 1.0070x over previous
import functools

import jax
import jax.numpy as jnp
from jax import lax
from jax.experimental import pallas as pl
from jax.experimental.pallas import tpu as pltpu
from jax.experimental.pallas import tpu_sc as plsc

N = 10000
M = 10000
NNZ = 160000
D = 256
DH = 128
NC = 2
NT = 16
K = 128
PT = 10240
CH = PT // K
PB = 20
NNZ_PAD = NT * PT
SACC = 10240
ZR = 32
ROWS = 1000
HPT = NNZ // NT


def _mesh():
    return plsc.VectorSubcoreMesh(core_axis_name="c", subcore_axis_name="s")


@functools.cache
def _build_sc_degrees():
    @functools.partial(
        pl.kernel,
        out_type=(
            jax.ShapeDtypeStruct((NT, N), jnp.float32),
            jax.ShapeDtypeStruct((NT, M), jnp.float32),
        ),
        mesh=_mesh(),
        compiler_params=pltpu.CompilerParams(needs_layout_passes=False),
        scratch_types=[
            pltpu.VMEM((HPT,), jnp.int32),
            pltpu.VMEM((N,), jnp.float32),
        ],
    )
    def sc_degrees(node_hbm, edge_hbm, dvp_hbm, dep_hbm, idx_v, hist_v):
        c = lax.axis_index("c")
        t = lax.axis_index("s")

        def do_hist(src_hbm, out_hbm):
            pltpu.sync_copy(src_hbm.at[pl.ds(t * HPT, HPT)], idx_v)

            def zero(i, carry):
                hist_v[pl.ds(i * 16, 16)] = jnp.zeros((16,), jnp.float32)
                return carry

            lax.fori_loop(0, N // 16, zero, 0)

            ones = jnp.ones((16,), jnp.float32)

            def acc(i, carry):
                idx = idx_v[pl.ds(i * 16, 16)]
                plsc.addupdate_scatter(hist_v, [idx], ones)
                return carry

            lax.fori_loop(0, HPT // 16, acc, 0)
            pltpu.sync_copy(hist_v, out_hbm.at[t])

        @pl.when(c == 0)
        def _():
            do_hist(node_hbm, dvp_hbm)

        @pl.when(c == 1)
        def _():
            do_hist(edge_hbm, dep_hbm)

    return sc_degrees


def _sc_degrees_call(node_idx, edge_idx):
    return _build_sc_degrees()(node_idx, edge_idx)


def _degrees_body(dvp_ref, dep_ref, a_ref, einv_ref):
    dv = jnp.sum(dvp_ref[...], axis=0, keepdims=True)
    de = jnp.sum(dep_ref[...], axis=0, keepdims=True)
    a_ref[...] = lax.rsqrt(dv)
    einv_ref[...] = 1.0 / de


def _tc_degrees(dvp, dep):
    return pl.pallas_call(
        _degrees_body,
        out_shape=[
            jax.ShapeDtypeStruct((1, N), jnp.float32),
            jax.ShapeDtypeStruct((1, M), jnp.float32),
        ],
    )(dvp, dep)


@functools.cache
def _build_sc_lmm():
    @functools.partial(
        pl.kernel,
        out_type=(
            jax.ShapeDtypeStruct((NC, SACC, DH), jnp.float32),
            jax.ShapeDtypeStruct((NC, SACC, DH), jnp.float32),
        ),
        mesh=_mesh(),
        compiler_params=pltpu.CompilerParams(needs_layout_passes=False),
        scratch_types=[
            pltpu.VMEM((2, PB, 2, K), jnp.int32),
            pltpu.VMEM((2, K, DH), jnp.float32),
            pltpu.VMEM((SACC // NT,), jnp.float32),
            pltpu.VMEM((ZR, DH), jnp.float32),
            pltpu.VMEM_SHARED((SACC, DH), jnp.float32),
            pltpu.SemaphoreType.DMA,
            pltpu.SemaphoreType.DMA,
        ],
    )
    def sc_lmm(table_hbm, pairsa_hbm, pairsb_hbm, scale_hbm,
               mid_hbm, out_hbm, pbuf, rows, ebuf, zbuf, acc, gsem0, gsem1):
        c = lax.axis_index("c")
        t = lax.axis_index("s")
        gsems = (gsem0, gsem1)
        rpt = SACC // NT
        base = t * rpt
        nblk = rpt // K

        def zset(i, carry):
            r = i // (DH // 16)
            col = (i % (DH // 16)) * 16
            zbuf[r, pl.ds(col, 16)] = jnp.zeros((16,), jnp.float32)
            return carry

        lax.fori_loop(0, ZR * (DH // 16), zset, 0)
        pltpu.sync_copy(scale_hbm.at[pl.ds(base, rpt)], ebuf)

        def zero_acc():
            def zfire(i, carry):
                pltpu.async_copy(zbuf, acc.at[pl.ds(base + i * ZR, ZR)], gsem0)
                return carry

            lax.fori_loop(0, rpt // ZR, zfire, 0)

            def zwait(i, carry):
                pltpu.make_async_copy(
                    zbuf, acc.at[pl.ds(base + i * ZR, ZR)], gsem0
                ).wait()
                return carry

            lax.fori_loop(0, rpt // ZR, zwait, 0)

        def stage(tbl, pairs_hbm, dst_hbm, scaled):
            def load_block(b):
                pltpu.sync_copy(
                    pairs_hbm.at[pl.ds(t * CH + b * PB, PB)],
                    pbuf.at[lax.rem(b, 2)],
                )

            def gidx(j):
                return pbuf.at[lax.rem(j // PB, 2), lax.rem(j, PB), 0]

            def sidx(j):
                return pbuf.at[lax.rem(j // PB, 2), lax.rem(j, PB), 1]

            def fire_gather(slot, j):
                pltpu.async_copy(tbl.at[gidx(j)], rows.at[slot], gsems[slot])

            def wait_gather(slot, j):
                pltpu.make_async_copy(
                    tbl.at[gidx(j)], rows.at[slot], gsems[slot]
                ).wait()

            def scatter_add(slot, j):
                pltpu.sync_copy(rows.at[slot], acc.at[sidx(j)], add=True)

            load_block(0)
            fire_gather(0, 0)

            def body(jj, carry):
                j0 = 2 * jj
                j1 = j0 + 1
                fire_gather(1, j1)
                wait_gather(0, j0)
                scatter_add(0, j0)

                @pl.when(jj != CH // 2 - 1)
                def _():
                    @pl.when(lax.rem(j0 + 2, PB) == 0)
                    def _():
                        load_block((j0 + 2) // PB)

                    fire_gather(0, j0 + 2)

                wait_gather(1, j1)
                scatter_add(1, j1)
                return carry

            lax.fori_loop(0, CH // 2, body, 0)
            plsc.subcore_barrier()

            if not scaled:
                pltpu.sync_copy(
                    acc.at[pl.ds(base, rpt)],
                    dst_hbm.at[c].at[pl.ds(base, rpt)],
                )
            else:
                pltpu.async_copy(acc.at[pl.ds(base, K)], rows.at[0], gsem0)
                dnums = lax.GatherDimensionNumbers(
                    offset_dims=(), collapsed_slice_dims=(0,),
                    start_index_map=(0,))
                for b in range(nblk):
                    sl = b % 2
                    pltpu.make_async_copy(
                        acc.at[pl.ds(base + b * K, K)], rows.at[sl], gsems[sl]
                    ).wait()
                    if b + 1 < nblk:
                        pltpu.async_copy(
                            acc.at[pl.ds(base + (b + 1) * K, K)],
                            rows.at[1 - sl],
                            gsems[1 - sl],
                        )

                    def scale_group(g, carry):
                        e16 = ebuf[pl.ds(b * K + g * 16, 16)]
                        for i in range(16):
                            r = g * 16 + i
                            e = lax.gather(
                                e16,
                                jnp.full((16, 1), i, jnp.int32),
                                dnums,
                                slice_sizes=(1,),
                                mode=lax.GatherScatterMode.PROMISE_IN_BOUNDS,
                            )
                            for q in range(DH // 16):
                                rows[sl, r, pl.ds(q * 16, 16)] = (
                                    rows[sl, r, pl.ds(q * 16, 16)] * e
                                )
                        return carry

                    lax.fori_loop(0, K // 16, scale_group, 0)
                    pltpu.sync_copy(
                        rows.at[sl],
                        dst_hbm.at[c].at[pl.ds(base + b * K, K)],
                    )

        zero_acc()
        plsc.subcore_barrier()
        stage(table_hbm.at[c], pairsa_hbm, mid_hbm, True)
        zero_acc()
        plsc.subcore_barrier()
        stage(mid_hbm.at[c], pairsb_hbm, out_hbm, False)

    return sc_lmm


def _sc_lmm_call(table, pairs_a, pairs_b, scale_pad):
    return _build_sc_lmm()(table, pairs_a, pairs_b, scale_pad)


def _mm1_body(x_ref, w_ref, b_ref, a_ref, out_ref):
    y = lax.dot_general(
        x_ref[...], w_ref[...], (((1,), (1,)), ((), ())),
        preferred_element_type=jnp.float32,
    )
    out_ref[...] = ((y + b_ref[...]) * a_ref[...])[None]


def _tc_mm1(X, W1, b1, acol):
    return pl.pallas_call(
        _mm1_body,
        grid=(NC, N // ROWS),
        in_specs=[
            pl.BlockSpec((ROWS, D), lambda c, i: (i, 0)),
            pl.BlockSpec((DH, D), lambda c, i: (c, 0)),
            pl.BlockSpec((1, DH), lambda c, i: (0, c)),
            pl.BlockSpec((ROWS, 1), lambda c, i: (i, 0)),
        ],
        out_specs=pl.BlockSpec((1, ROWS, DH), lambda c, i: (c, i, 0)),
        out_shape=jax.ShapeDtypeStruct((NC, SACC, DH), jnp.float32),
    )(X, W1, b1.reshape(1, D), acol)


def _mm2_body(z_ref, a_ref, w_ref, b_ref, out_ref):
    k = pl.program_id(2)
    a = a_ref[...]
    h = jnp.maximum(z_ref[0] * a, 0.0)
    p = lax.dot_general(
        h, w_ref[...], (((1,), (1,)), ((), ())),
        preferred_element_type=jnp.float32,
    )

    @pl.when(k == 0)
    def _():
        out_ref[...] = p[None]

    @pl.when(k == 1)
    def _():
        out_ref[...] = ((out_ref[0] + p + b_ref[...]) * a)[None]


def _tc_mm2(Zv, W2, b2, acol):
    return pl.pallas_call(
        _mm2_body,
        grid=(NC, N // ROWS, NC),
        in_specs=[
            pl.BlockSpec((1, ROWS, DH), lambda c, i, k: (k, i, 0)),
            pl.BlockSpec((ROWS, 1), lambda c, i, k: (i, 0)),
            pl.BlockSpec((DH, DH), lambda c, i, k: (c, k)),
            pl.BlockSpec((1, DH), lambda c, i, k: (0, c)),
        ],
        out_specs=pl.BlockSpec((1, ROWS, DH), lambda c, i, k: (c, i, 0)),
        out_shape=jax.ShapeDtypeStruct((NC, SACC, DH), jnp.float32),
    )(Zv, acol, W2, b2.reshape(1, D))


def _final_body(z_ref, a_ref, out_ref):
    out_ref[...] = z_ref[0] * a_ref[...]


def _tc_final(Zv, acol):
    return pl.pallas_call(
        _final_body,
        grid=(NC, N // ROWS),
        in_specs=[
            pl.BlockSpec((1, ROWS, DH), lambda c, i: (c, i, 0)),
            pl.BlockSpec((ROWS, 1), lambda c, i: (i, 0)),
        ],
        out_specs=pl.BlockSpec((ROWS, DH), lambda c, i: (i, c)),
        out_shape=jax.ShapeDtypeStruct((N, D), jnp.float32),
    )(Zv, acol)


def _pack_pairs(gidx, sidx, gmod, spad_base):
    npad = NNZ_PAD - NNZ
    fill = jnp.arange(npad, dtype=jnp.int32)
    g = jnp.concatenate([gidx.astype(jnp.int32), fill % gmod])
    s = jnp.concatenate([sidx.astype(jnp.int32),
                         spad_base + fill % (SACC - spad_base)])
    g3 = g.reshape(NT, CH, 1, K)
    s3 = s.reshape(NT, CH, 1, K)
    return jnp.concatenate([g3, s3], axis=2).reshape(NT * CH, 2, K)


def kernel(X, W1, b1, W2, b2, node_idx, edge_idx):
    pairs_ne = _pack_pairs(node_idx, edge_idx, N, M)
    pairs_en = _pack_pairs(edge_idx, node_idx, M, N)

    dvp, dep = _sc_degrees_call(node_idx, edge_idx)
    a_row, einv_row = _tc_degrees(dvp, dep)
    acol = a_row.reshape(N, 1)
    einv_pad = jnp.concatenate(
        [einv_row.reshape(M), jnp.ones((SACC - M,), jnp.float32)])

    y1 = _tc_mm1(X, W1, b1, acol)
    _, zv = _sc_lmm_call(y1, pairs_ne, pairs_en, einv_pad)
    y2 = _tc_mm2(zv, W2, b2, acol)
    _, zv2 = _sc_lmm_call(y2, pairs_ne, pairs_en, einv_pad)
    return _tc_final(zv2, acol)


# speedup vs baseline: 1.0312x; 1.0240x over previous
import functools

import jax
import jax.numpy as jnp
from jax import lax
from jax.experimental import pallas as pl
from jax.experimental.pallas import tpu as pltpu
from jax.experimental.pallas import tpu_sc as plsc

N = 10000
M = 10000
NNZ = 160000
D = 256
DH = 128
NC = 2
NT = 16
K = 128
PT = 10240
CH = PT // K
PB = 20
NNZ_PAD = NT * PT
SACC = 10240
ZR = 32
ROWS = 1000
HPT = NNZ // NT


def _mesh():
    return plsc.VectorSubcoreMesh(core_axis_name="c", subcore_axis_name="s")


@functools.cache
def _build_sc_degrees():
    @functools.partial(
        pl.kernel,
        out_type=(
            jax.ShapeDtypeStruct((NT, N), jnp.float32),
            jax.ShapeDtypeStruct((NT, M), jnp.float32),
        ),
        mesh=_mesh(),
        compiler_params=pltpu.CompilerParams(needs_layout_passes=False),
        scratch_types=[
            pltpu.VMEM((HPT,), jnp.int32),
            pltpu.VMEM((N,), jnp.float32),
        ],
    )
    def sc_degrees(node_hbm, edge_hbm, dvp_hbm, dep_hbm, idx_v, hist_v):
        c = lax.axis_index("c")
        t = lax.axis_index("s")

        def do_hist(src_hbm, out_hbm):
            pltpu.sync_copy(src_hbm.at[pl.ds(t * HPT, HPT)], idx_v)

            def zero(i, carry):
                hist_v[pl.ds(i * 16, 16)] = jnp.zeros((16,), jnp.float32)
                return carry

            lax.fori_loop(0, N // 16, zero, 0)

            ones = jnp.ones((16,), jnp.float32)

            def acc(i, carry):
                idx = idx_v[pl.ds(i * 16, 16)]
                plsc.addupdate_scatter(hist_v, [idx], ones)
                return carry

            lax.fori_loop(0, HPT // 16, acc, 0)
            pltpu.sync_copy(hist_v, out_hbm.at[t])

        @pl.when(c == 0)
        def _():
            do_hist(node_hbm, dvp_hbm)

        @pl.when(c == 1)
        def _():
            do_hist(edge_hbm, dep_hbm)

    return sc_degrees


def _sc_degrees_call(node_idx, edge_idx):
    return _build_sc_degrees()(node_idx, edge_idx)


def _degrees_body(dvp_ref, dep_ref, a_ref, einv_ref):
    dv = jnp.sum(dvp_ref[...], axis=0, keepdims=True)
    de = jnp.sum(dep_ref[...], axis=0, keepdims=True)
    a_ref[...] = lax.rsqrt(dv)
    einv_ref[...] = 1.0 / de


def _tc_degrees(dvp, dep):
    return pl.pallas_call(
        _degrees_body,
        out_shape=[
            jax.ShapeDtypeStruct((1, N), jnp.float32),
            jax.ShapeDtypeStruct((1, M), jnp.float32),
        ],
    )(dvp, dep)


@functools.cache
def _build_sc_lmm():
    @functools.partial(
        pl.kernel,
        out_type=(
            jax.ShapeDtypeStruct((NC, SACC, DH), jnp.float32),
            jax.ShapeDtypeStruct((NC, SACC, DH), jnp.float32),
        ),
        mesh=_mesh(),
        compiler_params=pltpu.CompilerParams(needs_layout_passes=False),
        scratch_types=[
            pltpu.VMEM((2, PB, 2, K), jnp.int32),
            pltpu.VMEM((2, K, DH), jnp.float32),
            pltpu.VMEM((SACC // NT,), jnp.float32),
            pltpu.VMEM((ZR, DH), jnp.float32),
            pltpu.VMEM_SHARED((SACC, DH), jnp.float32),
            pltpu.SemaphoreType.DMA,
            pltpu.SemaphoreType.DMA,
        ],
    )
    def sc_lmm(table_hbm, pairsa_hbm, pairsb_hbm, scale_hbm,
               mid_hbm, out_hbm, pbuf, rows, ebuf, zbuf, acc, gsem0, gsem1):
        c = lax.axis_index("c")
        t = lax.axis_index("s")
        gsems = (gsem0, gsem1)
        rpt = SACC // NT
        base = t * rpt
        nblk = rpt // K

        def zset(i, carry):
            r = i // (DH // 16)
            col = (i % (DH // 16)) * 16
            zbuf[r, pl.ds(col, 16)] = jnp.zeros((16,), jnp.float32)
            return carry

        lax.fori_loop(0, ZR * (DH // 16), zset, 0)
        pltpu.sync_copy(scale_hbm.at[pl.ds(base, rpt)], ebuf)

        def zero_acc():
            def zfire(i, carry):
                pltpu.async_copy(zbuf, acc.at[pl.ds(base + i * ZR, ZR)], gsem0)
                return carry

            lax.fori_loop(0, rpt // ZR, zfire, 0)

            def zwait(i, carry):
                pltpu.make_async_copy(
                    zbuf, acc.at[pl.ds(base + i * ZR, ZR)], gsem0
                ).wait()
                return carry

            lax.fori_loop(0, rpt // ZR, zwait, 0)

        def stage(tbl, pairs_hbm, dst_hbm, scaled):
            def load_block(b):
                pltpu.sync_copy(
                    pairs_hbm.at[pl.ds(t * CH + b * PB, PB)],
                    pbuf.at[lax.rem(b, 2)],
                )

            def gidx(j):
                return pbuf.at[lax.rem(j // PB, 2), lax.rem(j, PB), 0]

            def sidx(j):
                return pbuf.at[lax.rem(j // PB, 2), lax.rem(j, PB), 1]

            def fire_gather(slot, j):
                pltpu.async_copy(tbl.at[gidx(j)], rows.at[slot], gsems[slot])

            def wait_gather(slot, j):
                pltpu.make_async_copy(
                    tbl.at[gidx(j)], rows.at[slot], gsems[slot]
                ).wait()

            def scatter_add(slot, j):
                pltpu.sync_copy(rows.at[slot], acc.at[sidx(j)], add=True)

            load_block(0)
            fire_gather(0, 0)

            def body(jj, carry):
                j0 = 2 * jj
                j1 = j0 + 1
                fire_gather(1, j1)
                wait_gather(0, j0)
                scatter_add(0, j0)

                @pl.when(jj != CH // 2 - 1)
                def _():
                    @pl.when(lax.rem(j0 + 2, PB) == 0)
                    def _():
                        load_block((j0 + 2) // PB)

                    fire_gather(0, j0 + 2)

                wait_gather(1, j1)
                scatter_add(1, j1)
                return carry

            lax.fori_loop(0, CH // 2, body, 0)
            plsc.subcore_barrier()

            if not scaled:
                pltpu.sync_copy(
                    acc.at[pl.ds(base, rpt)],
                    dst_hbm.at[c].at[pl.ds(base, rpt)],
                )
            else:
                pltpu.async_copy(acc.at[pl.ds(base, K)], rows.at[0], gsem0)
                dnums = lax.GatherDimensionNumbers(
                    offset_dims=(), collapsed_slice_dims=(0,),
                    start_index_map=(0,))
                for b in range(nblk):
                    sl = b % 2
                    pltpu.make_async_copy(
                        acc.at[pl.ds(base + b * K, K)], rows.at[sl], gsems[sl]
                    ).wait()
                    if b + 1 < nblk:
                        pltpu.async_copy(
                            acc.at[pl.ds(base + (b + 1) * K, K)],
                            rows.at[1 - sl],
                            gsems[1 - sl],
                        )

                    def scale_group(g, carry):
                        e16 = ebuf[pl.ds(b * K + g * 16, 16)]
                        for i in range(16):
                            r = g * 16 + i
                            e = lax.gather(
                                e16,
                                jnp.full((16, 1), i, jnp.int32),
                                dnums,
                                slice_sizes=(1,),
                                mode=lax.GatherScatterMode.PROMISE_IN_BOUNDS,
                            )
                            for q in range(DH // 16):
                                rows[sl, r, pl.ds(q * 16, 16)] = (
                                    rows[sl, r, pl.ds(q * 16, 16)] * e
                                )
                        return carry

                    lax.fori_loop(0, K // 16, scale_group, 0)
                    pltpu.sync_copy(
                        rows.at[sl],
                        dst_hbm.at[c].at[pl.ds(base + b * K, K)],
                    )

        zero_acc()
        plsc.subcore_barrier()
        stage(table_hbm.at[c], pairsa_hbm, mid_hbm, True)
        zero_acc()
        plsc.subcore_barrier()
        stage(mid_hbm.at[c], pairsb_hbm, out_hbm, False)

    return sc_lmm


def _sc_lmm_call(table, pairs_a, pairs_b, scale_pad):
    return _build_sc_lmm()(table, pairs_a, pairs_b, scale_pad)


def _mm1_body(x_ref, w_ref, b_ref, a_ref, out_ref):
    y = lax.dot_general(
        x_ref[...], w_ref[...], (((1,), (1,)), ((), ())),
        preferred_element_type=jnp.float32,
    )
    out_ref[...] = ((y + b_ref[...]) * a_ref[...])[None]


def _tc_mm1(X, W1, b1, acol):
    return pl.pallas_call(
        _mm1_body,
        grid=(NC, N // ROWS),
        in_specs=[
            pl.BlockSpec((ROWS, D), lambda c, i: (i, 0)),
            pl.BlockSpec((DH, D), lambda c, i: (c, 0)),
            pl.BlockSpec((1, DH), lambda c, i: (0, c)),
            pl.BlockSpec((ROWS, 1), lambda c, i: (i, 0)),
        ],
        out_specs=pl.BlockSpec((1, ROWS, DH), lambda c, i: (c, i, 0)),
        out_shape=jax.ShapeDtypeStruct((NC, SACC, DH), jnp.float32),
    )(X, W1, b1.reshape(1, D), acol)


def _mm2_body(z_ref, a_ref, w_ref, b_ref, out_ref):
    a = a_ref[...]
    h0 = jnp.maximum(z_ref[0] * a, 0.0)
    h1 = jnp.maximum(z_ref[1] * a, 0.0)
    p = lax.dot_general(
        h0, w_ref[:, :DH], (((1,), (1,)), ((), ())),
        preferred_element_type=jnp.float32,
    ) + lax.dot_general(
        h1, w_ref[:, DH:], (((1,), (1,)), ((), ())),
        preferred_element_type=jnp.float32,
    )
    out_ref[...] = ((p + b_ref[...]) * a)[None]


def _tc_mm2(Zv, W2, b2, acol):
    return pl.pallas_call(
        _mm2_body,
        grid=(NC, N // ROWS),
        in_specs=[
            pl.BlockSpec((NC, ROWS, DH), lambda c, i: (0, i, 0)),
            pl.BlockSpec((ROWS, 1), lambda c, i: (i, 0)),
            pl.BlockSpec((DH, D), lambda c, i: (c, 0)),
            pl.BlockSpec((1, DH), lambda c, i: (0, c)),
        ],
        out_specs=pl.BlockSpec((1, ROWS, DH), lambda c, i: (c, i, 0)),
        out_shape=jax.ShapeDtypeStruct((NC, SACC, DH), jnp.float32),
    )(Zv, acol, W2, b2.reshape(1, D))


def _final_body(z_ref, a_ref, out_ref):
    out_ref[...] = z_ref[0] * a_ref[...]


def _tc_final(Zv, acol):
    return pl.pallas_call(
        _final_body,
        grid=(NC, N // ROWS),
        in_specs=[
            pl.BlockSpec((1, ROWS, DH), lambda c, i: (c, i, 0)),
            pl.BlockSpec((ROWS, 1), lambda c, i: (i, 0)),
        ],
        out_specs=pl.BlockSpec((ROWS, DH), lambda c, i: (i, c)),
        out_shape=jax.ShapeDtypeStruct((N, D), jnp.float32),
    )(Zv, acol)


def _pack_pairs(gidx, sidx, gmod, spad_base):
    npad = NNZ_PAD - NNZ
    fill = jnp.arange(npad, dtype=jnp.int32)
    g = jnp.concatenate([gidx.astype(jnp.int32), fill % gmod])
    s = jnp.concatenate([sidx.astype(jnp.int32),
                         spad_base + fill % (SACC - spad_base)])
    g3 = g.reshape(NT, CH, 1, K)
    s3 = s.reshape(NT, CH, 1, K)
    return jnp.concatenate([g3, s3], axis=2).reshape(NT * CH, 2, K)


def kernel(X, W1, b1, W2, b2, node_idx, edge_idx):
    pairs_ne = _pack_pairs(node_idx, edge_idx, N, M)
    pairs_en = _pack_pairs(edge_idx, node_idx, M, N)

    dvp, dep = _sc_degrees_call(node_idx, edge_idx)
    a_row, einv_row = _tc_degrees(dvp, dep)
    acol = a_row.reshape(N, 1)
    einv_pad = jnp.concatenate(
        [einv_row.reshape(M), jnp.ones((SACC - M,), jnp.float32)])

    y1 = _tc_mm1(X, W1, b1, acol)
    _, zv = _sc_lmm_call(y1, pairs_ne, pairs_en, einv_pad)
    y2 = _tc_mm2(zv, W2, b2, acol)
    _, zv2 = _sc_lmm_call(y2, pairs_ne, pairs_en, einv_pad)
    return _tc_final(zv2, acol)


# speedup vs baseline: 1.0723x; 1.0399x over previous
import functools

import jax
import jax.numpy as jnp
from jax import lax
from jax.experimental import pallas as pl
from jax.experimental.pallas import tpu as pltpu
from jax.experimental.pallas import tpu_sc as plsc

N = 10000
M = 10000
NNZ = 160000
D = 256
DH = 128
NC = 2
NT = 16
K = 128
PT = 10240
CH = PT // K
PB = 20
NNZ_PAD = NT * PT
SACC = 10240
ZR = 32
ROWS = 2000
HPT = NNZ // NT


def _mesh():
    return plsc.VectorSubcoreMesh(core_axis_name="c", subcore_axis_name="s")


@functools.cache
def _build_sc_degrees():
    @functools.partial(
        pl.kernel,
        out_type=(
            jax.ShapeDtypeStruct((NT, N), jnp.float32),
            jax.ShapeDtypeStruct((NT, M), jnp.float32),
        ),
        mesh=_mesh(),
        compiler_params=pltpu.CompilerParams(needs_layout_passes=False),
        scratch_types=[
            pltpu.VMEM((HPT,), jnp.int32),
            pltpu.VMEM((N,), jnp.float32),
        ],
    )
    def sc_degrees(node_hbm, edge_hbm, dvp_hbm, dep_hbm, idx_v, hist_v):
        c = lax.axis_index("c")
        t = lax.axis_index("s")

        def do_hist(src_hbm, out_hbm):
            pltpu.sync_copy(src_hbm.at[pl.ds(t * HPT, HPT)], idx_v)

            def zero(i, carry):
                hist_v[pl.ds(i * 16, 16)] = jnp.zeros((16,), jnp.float32)
                return carry

            lax.fori_loop(0, N // 16, zero, 0)

            ones = jnp.ones((16,), jnp.float32)

            def acc(i, carry):
                idx = idx_v[pl.ds(i * 16, 16)]
                plsc.addupdate_scatter(hist_v, [idx], ones)
                return carry

            lax.fori_loop(0, HPT // 16, acc, 0)
            pltpu.sync_copy(hist_v, out_hbm.at[t])

        @pl.when(c == 0)
        def _():
            do_hist(node_hbm, dvp_hbm)

        @pl.when(c == 1)
        def _():
            do_hist(edge_hbm, dep_hbm)

    return sc_degrees


def _sc_degrees_call(node_idx, edge_idx):
    return _build_sc_degrees()(node_idx, edge_idx)


def _degrees_body(dvp_ref, dep_ref, a_ref, einv_ref):
    dv = jnp.sum(dvp_ref[...], axis=0, keepdims=True)
    de = jnp.sum(dep_ref[...], axis=0, keepdims=True)
    a_ref[...] = lax.rsqrt(dv)
    einv_ref[...] = 1.0 / de


def _tc_degrees(dvp, dep):
    return pl.pallas_call(
        _degrees_body,
        out_shape=[
            jax.ShapeDtypeStruct((1, N), jnp.float32),
            jax.ShapeDtypeStruct((1, M), jnp.float32),
        ],
    )(dvp, dep)


@functools.cache
def _build_sc_lmm():
    @functools.partial(
        pl.kernel,
        out_type=(
            jax.ShapeDtypeStruct((NC, SACC, DH), jnp.float32),
            jax.ShapeDtypeStruct((NC, SACC, DH), jnp.float32),
        ),
        mesh=_mesh(),
        compiler_params=pltpu.CompilerParams(needs_layout_passes=False),
        scratch_types=[
            pltpu.VMEM((2, PB, 2, K), jnp.int32),
            pltpu.VMEM((2, K, DH), jnp.float32),
            pltpu.VMEM((SACC // NT,), jnp.float32),
            pltpu.VMEM((ZR, DH), jnp.float32),
            pltpu.VMEM_SHARED((SACC, DH), jnp.float32),
            pltpu.SemaphoreType.DMA,
            pltpu.SemaphoreType.DMA,
        ],
    )
    def sc_lmm(table_hbm, pairsa_hbm, pairsb_hbm, scale_hbm,
               mid_hbm, out_hbm, pbuf, rows, ebuf, zbuf, acc, gsem0, gsem1):
        c = lax.axis_index("c")
        t = lax.axis_index("s")
        gsems = (gsem0, gsem1)
        rpt = SACC // NT
        base = t * rpt
        nblk = rpt // K

        def zset(i, carry):
            r = i // (DH // 16)
            col = (i % (DH // 16)) * 16
            zbuf[r, pl.ds(col, 16)] = jnp.zeros((16,), jnp.float32)
            return carry

        lax.fori_loop(0, ZR * (DH // 16), zset, 0)
        pltpu.sync_copy(scale_hbm.at[pl.ds(base, rpt)], ebuf)

        def zero_acc():
            def zfire(i, carry):
                pltpu.async_copy(zbuf, acc.at[pl.ds(base + i * ZR, ZR)], gsem0)
                return carry

            lax.fori_loop(0, rpt // ZR, zfire, 0)

            def zwait(i, carry):
                pltpu.make_async_copy(
                    zbuf, acc.at[pl.ds(base + i * ZR, ZR)], gsem0
                ).wait()
                return carry

            lax.fori_loop(0, rpt // ZR, zwait, 0)

        def stage(tbl, pairs_hbm, dst_hbm, scaled):
            def load_block(b):
                pltpu.sync_copy(
                    pairs_hbm.at[pl.ds(t * CH + b * PB, PB)],
                    pbuf.at[lax.rem(b, 2)],
                )

            def gidx(j):
                return pbuf.at[lax.rem(j // PB, 2), lax.rem(j, PB), 0]

            def sidx(j):
                return pbuf.at[lax.rem(j // PB, 2), lax.rem(j, PB), 1]

            def fire_gather(slot, j):
                pltpu.async_copy(tbl.at[gidx(j)], rows.at[slot], gsems[slot])

            def wait_gather(slot, j):
                pltpu.make_async_copy(
                    tbl.at[gidx(j)], rows.at[slot], gsems[slot]
                ).wait()

            def scatter_add(slot, j):
                pltpu.sync_copy(rows.at[slot], acc.at[sidx(j)], add=True)

            load_block(0)
            fire_gather(0, 0)

            def body(jj, carry):
                j0 = 2 * jj
                j1 = j0 + 1
                fire_gather(1, j1)
                wait_gather(0, j0)
                scatter_add(0, j0)

                @pl.when(jj != CH // 2 - 1)
                def _():
                    @pl.when(lax.rem(j0 + 2, PB) == 0)
                    def _():
                        load_block((j0 + 2) // PB)

                    fire_gather(0, j0 + 2)

                wait_gather(1, j1)
                scatter_add(1, j1)
                return carry

            lax.fori_loop(0, CH // 2, body, 0)
            plsc.subcore_barrier()

            if not scaled:
                pltpu.sync_copy(
                    acc.at[pl.ds(base, rpt)],
                    dst_hbm.at[c].at[pl.ds(base, rpt)],
                )
            else:
                pltpu.async_copy(acc.at[pl.ds(base, K)], rows.at[0], gsem0)
                dnums = lax.GatherDimensionNumbers(
                    offset_dims=(), collapsed_slice_dims=(0,),
                    start_index_map=(0,))
                for b in range(nblk):
                    sl = b % 2
                    pltpu.make_async_copy(
                        acc.at[pl.ds(base + b * K, K)], rows.at[sl], gsems[sl]
                    ).wait()
                    if b + 1 < nblk:
                        pltpu.async_copy(
                            acc.at[pl.ds(base + (b + 1) * K, K)],
                            rows.at[1 - sl],
                            gsems[1 - sl],
                        )

                    def scale_group(g, carry):
                        e16 = ebuf[pl.ds(b * K + g * 16, 16)]
                        for i in range(16):
                            r = g * 16 + i
                            e = lax.gather(
                                e16,
                                jnp.full((16, 1), i, jnp.int32),
                                dnums,
                                slice_sizes=(1,),
                                mode=lax.GatherScatterMode.PROMISE_IN_BOUNDS,
                            )
                            for q in range(DH // 16):
                                rows[sl, r, pl.ds(q * 16, 16)] = (
                                    rows[sl, r, pl.ds(q * 16, 16)] * e
                                )
                        return carry

                    lax.fori_loop(0, K // 16, scale_group, 0)
                    pltpu.sync_copy(
                        rows.at[sl],
                        dst_hbm.at[c].at[pl.ds(base + b * K, K)],
                    )

        zero_acc()
        plsc.subcore_barrier()
        stage(table_hbm.at[c], pairsa_hbm, mid_hbm, True)
        zero_acc()
        plsc.subcore_barrier()
        stage(mid_hbm.at[c], pairsb_hbm, out_hbm, False)

    return sc_lmm


def _sc_lmm_call(table, pairs_a, pairs_b, scale_pad):
    return _build_sc_lmm()(table, pairs_a, pairs_b, scale_pad)


def _mm1_body(x_ref, w_ref, b_ref, a_ref, out_ref):
    y = lax.dot_general(
        x_ref[...], w_ref[...], (((1,), (1,)), ((), ())),
        preferred_element_type=jnp.float32,
    )
    out_ref[...] = ((y + b_ref[...]) * a_ref[...])[None]


def _tc_mm1(X, W1, b1, acol):
    return pl.pallas_call(
        _mm1_body,
        grid=(NC, N // ROWS),
        in_specs=[
            pl.BlockSpec((ROWS, D), lambda c, i: (i, 0)),
            pl.BlockSpec((DH, D), lambda c, i: (c, 0)),
            pl.BlockSpec((1, DH), lambda c, i: (0, c)),
            pl.BlockSpec((ROWS, 1), lambda c, i: (i, 0)),
        ],
        out_specs=pl.BlockSpec((1, ROWS, DH), lambda c, i: (c, i, 0)),
        out_shape=jax.ShapeDtypeStruct((NC, SACC, DH), jnp.float32),
    )(X, W1, b1.reshape(1, D), acol)


def _mm2_body(z_ref, a_ref, w_ref, b_ref, out_ref):
    a = a_ref[...]
    h0 = jnp.maximum(z_ref[0] * a, 0.0)
    h1 = jnp.maximum(z_ref[1] * a, 0.0)
    p = lax.dot_general(
        h0, w_ref[:, :DH], (((1,), (1,)), ((), ())),
        preferred_element_type=jnp.float32,
    ) + lax.dot_general(
        h1, w_ref[:, DH:], (((1,), (1,)), ((), ())),
        preferred_element_type=jnp.float32,
    )
    out_ref[...] = ((p + b_ref[...]) * a)[None]


def _tc_mm2(Zv, W2, b2, acol):
    return pl.pallas_call(
        _mm2_body,
        grid=(NC, N // ROWS),
        in_specs=[
            pl.BlockSpec((NC, ROWS, DH), lambda c, i: (0, i, 0)),
            pl.BlockSpec((ROWS, 1), lambda c, i: (i, 0)),
            pl.BlockSpec((DH, D), lambda c, i: (c, 0)),
            pl.BlockSpec((1, DH), lambda c, i: (0, c)),
        ],
        out_specs=pl.BlockSpec((1, ROWS, DH), lambda c, i: (c, i, 0)),
        out_shape=jax.ShapeDtypeStruct((NC, SACC, DH), jnp.float32),
    )(Zv, acol, W2, b2.reshape(1, D))


def _final_body(z_ref, a_ref, out_ref):
    out_ref[...] = z_ref[0] * a_ref[...]


def _tc_final(Zv, acol):
    return pl.pallas_call(
        _final_body,
        grid=(NC, N // ROWS),
        in_specs=[
            pl.BlockSpec((1, ROWS, DH), lambda c, i: (c, i, 0)),
            pl.BlockSpec((ROWS, 1), lambda c, i: (i, 0)),
        ],
        out_specs=pl.BlockSpec((ROWS, DH), lambda c, i: (i, c)),
        out_shape=jax.ShapeDtypeStruct((N, D), jnp.float32),
    )(Zv, acol)


def _pack_pairs(gidx, sidx, gmod, spad_base):
    npad = NNZ_PAD - NNZ
    fill = jnp.arange(npad, dtype=jnp.int32)
    g = jnp.concatenate([gidx.astype(jnp.int32), fill % gmod])
    s = jnp.concatenate([sidx.astype(jnp.int32),
                         spad_base + fill % (SACC - spad_base)])
    g3 = g.reshape(NT, CH, 1, K)
    s3 = s.reshape(NT, CH, 1, K)
    return jnp.concatenate([g3, s3], axis=2).reshape(NT * CH, 2, K)


def kernel(X, W1, b1, W2, b2, node_idx, edge_idx):
    pairs_ne = _pack_pairs(node_idx, edge_idx, N, M)
    pairs_en = _pack_pairs(edge_idx, node_idx, M, N)

    dvp, dep = _sc_degrees_call(node_idx, edge_idx)
    a_row, einv_row = _tc_degrees(dvp, dep)
    acol = a_row.reshape(N, 1)
    einv_pad = jnp.concatenate(
        [einv_row.reshape(M), jnp.ones((SACC - M,), jnp.float32)])

    y1 = _tc_mm1(X, W1, b1, acol)
    _, zv = _sc_lmm_call(y1, pairs_ne, pairs_en, einv_pad)
    y2 = _tc_mm2(zv, W2, b2, acol)
    _, zv2 = _sc_lmm_call(y2, pairs_ne, pairs_en, einv_pad)
    return _tc_final(zv2, acol)


# speedup vs baseline: 1.0805x; 1.0077x over previous
import functools

import jax
import jax.numpy as jnp
from jax import lax
from jax.experimental import pallas as pl
from jax.experimental.pallas import tpu as pltpu
from jax.experimental.pallas import tpu_sc as plsc

N = 10000
M = 10000
NNZ = 160000
D = 256
DH = 128
NC = 2
NT = 16
K = 128
PT = 10240
CH = PT // K
PB = 20
NNZ_PAD = NT * PT
SACC = 10240
ZR = 32
ROWS = 5000
HPT = NNZ // NT


def _mesh():
    return plsc.VectorSubcoreMesh(core_axis_name="c", subcore_axis_name="s")


@functools.cache
def _build_sc_degrees():
    @functools.partial(
        pl.kernel,
        out_type=(
            jax.ShapeDtypeStruct((NT, N), jnp.float32),
            jax.ShapeDtypeStruct((NT, M), jnp.float32),
        ),
        mesh=_mesh(),
        compiler_params=pltpu.CompilerParams(needs_layout_passes=False),
        scratch_types=[
            pltpu.VMEM((HPT,), jnp.int32),
            pltpu.VMEM((N,), jnp.float32),
        ],
    )
    def sc_degrees(node_hbm, edge_hbm, dvp_hbm, dep_hbm, idx_v, hist_v):
        c = lax.axis_index("c")
        t = lax.axis_index("s")

        def do_hist(src_hbm, out_hbm):
            pltpu.sync_copy(src_hbm.at[pl.ds(t * HPT, HPT)], idx_v)

            def zero(i, carry):
                hist_v[pl.ds(i * 16, 16)] = jnp.zeros((16,), jnp.float32)
                return carry

            lax.fori_loop(0, N // 16, zero, 0)

            ones = jnp.ones((16,), jnp.float32)

            def acc(i, carry):
                idx = idx_v[pl.ds(i * 16, 16)]
                plsc.addupdate_scatter(hist_v, [idx], ones)
                return carry

            lax.fori_loop(0, HPT // 16, acc, 0)
            pltpu.sync_copy(hist_v, out_hbm.at[t])

        @pl.when(c == 0)
        def _():
            do_hist(node_hbm, dvp_hbm)

        @pl.when(c == 1)
        def _():
            do_hist(edge_hbm, dep_hbm)

    return sc_degrees


def _sc_degrees_call(node_idx, edge_idx):
    return _build_sc_degrees()(node_idx, edge_idx)


def _degrees_body(dvp_ref, dep_ref, a_ref, einv_ref):
    dv = jnp.sum(dvp_ref[...], axis=0, keepdims=True)
    de = jnp.sum(dep_ref[...], axis=0, keepdims=True)
    a_ref[...] = lax.rsqrt(dv)
    einv_ref[...] = 1.0 / de


def _tc_degrees(dvp, dep):
    return pl.pallas_call(
        _degrees_body,
        out_shape=[
            jax.ShapeDtypeStruct((1, N), jnp.float32),
            jax.ShapeDtypeStruct((1, M), jnp.float32),
        ],
    )(dvp, dep)


@functools.cache
def _build_sc_lmm():
    @functools.partial(
        pl.kernel,
        out_type=(
            jax.ShapeDtypeStruct((NC, SACC, DH), jnp.float32),
            jax.ShapeDtypeStruct((NC, SACC, DH), jnp.float32),
        ),
        mesh=_mesh(),
        compiler_params=pltpu.CompilerParams(needs_layout_passes=False),
        scratch_types=[
            pltpu.VMEM((2, PB, 2, K), jnp.int32),
            pltpu.VMEM((2, K, DH), jnp.float32),
            pltpu.VMEM((SACC // NT,), jnp.float32),
            pltpu.VMEM((ZR, DH), jnp.float32),
            pltpu.VMEM_SHARED((SACC, DH), jnp.float32),
            pltpu.SemaphoreType.DMA,
            pltpu.SemaphoreType.DMA,
        ],
    )
    def sc_lmm(table_hbm, pairsa_hbm, pairsb_hbm, scale_hbm,
               mid_hbm, out_hbm, pbuf, rows, ebuf, zbuf, acc, gsem0, gsem1):
        c = lax.axis_index("c")
        t = lax.axis_index("s")
        gsems = (gsem0, gsem1)
        rpt = SACC // NT
        base = t * rpt
        nblk = rpt // K

        def zset(i, carry):
            r = i // (DH // 16)
            col = (i % (DH // 16)) * 16
            zbuf[r, pl.ds(col, 16)] = jnp.zeros((16,), jnp.float32)
            return carry

        lax.fori_loop(0, ZR * (DH // 16), zset, 0)
        pltpu.sync_copy(scale_hbm.at[pl.ds(base, rpt)], ebuf)

        def zero_acc():
            def zfire(i, carry):
                pltpu.async_copy(zbuf, acc.at[pl.ds(base + i * ZR, ZR)], gsem0)
                return carry

            lax.fori_loop(0, rpt // ZR, zfire, 0)

            def zwait(i, carry):
                pltpu.make_async_copy(
                    zbuf, acc.at[pl.ds(base + i * ZR, ZR)], gsem0
                ).wait()
                return carry

            lax.fori_loop(0, rpt // ZR, zwait, 0)

        def stage(tbl, pairs_hbm, dst_hbm, scaled):
            def load_block(b):
                pltpu.sync_copy(
                    pairs_hbm.at[pl.ds(t * CH + b * PB, PB)],
                    pbuf.at[lax.rem(b, 2)],
                )

            def gidx(j):
                return pbuf.at[lax.rem(j // PB, 2), lax.rem(j, PB), 0]

            def sidx(j):
                return pbuf.at[lax.rem(j // PB, 2), lax.rem(j, PB), 1]

            def fire_gather(slot, j):
                pltpu.async_copy(tbl.at[gidx(j)], rows.at[slot], gsems[slot])

            def wait_gather(slot, j):
                pltpu.make_async_copy(
                    tbl.at[gidx(j)], rows.at[slot], gsems[slot]
                ).wait()

            def scatter_add(slot, j):
                pltpu.sync_copy(rows.at[slot], acc.at[sidx(j)], add=True)

            load_block(0)
            fire_gather(0, 0)

            def body(jj, carry):
                j0 = 2 * jj
                j1 = j0 + 1
                fire_gather(1, j1)
                wait_gather(0, j0)
                scatter_add(0, j0)

                @pl.when(jj != CH // 2 - 1)
                def _():
                    @pl.when(lax.rem(j0 + 2, PB) == 0)
                    def _():
                        load_block((j0 + 2) // PB)

                    fire_gather(0, j0 + 2)

                wait_gather(1, j1)
                scatter_add(1, j1)
                return carry

            lax.fori_loop(0, CH // 2, body, 0)
            plsc.subcore_barrier()

            if not scaled:
                pltpu.sync_copy(
                    acc.at[pl.ds(base, rpt)],
                    dst_hbm.at[c].at[pl.ds(base, rpt)],
                )
            else:
                pltpu.async_copy(acc.at[pl.ds(base, K)], rows.at[0], gsem0)
                dnums = lax.GatherDimensionNumbers(
                    offset_dims=(), collapsed_slice_dims=(0,),
                    start_index_map=(0,))
                for b in range(nblk):
                    sl = b % 2
                    pltpu.make_async_copy(
                        acc.at[pl.ds(base + b * K, K)], rows.at[sl], gsems[sl]
                    ).wait()
                    if b + 1 < nblk:
                        pltpu.async_copy(
                            acc.at[pl.ds(base + (b + 1) * K, K)],
                            rows.at[1 - sl],
                            gsems[1 - sl],
                        )

                    def scale_group(g, carry):
                        e16 = ebuf[pl.ds(b * K + g * 16, 16)]
                        for i in range(16):
                            r = g * 16 + i
                            e = lax.gather(
                                e16,
                                jnp.full((16, 1), i, jnp.int32),
                                dnums,
                                slice_sizes=(1,),
                                mode=lax.GatherScatterMode.PROMISE_IN_BOUNDS,
                            )
                            for q in range(DH // 16):
                                rows[sl, r, pl.ds(q * 16, 16)] = (
                                    rows[sl, r, pl.ds(q * 16, 16)] * e
                                )
                        return carry

                    lax.fori_loop(0, K // 16, scale_group, 0)
                    pltpu.sync_copy(
                        rows.at[sl],
                        dst_hbm.at[c].at[pl.ds(base + b * K, K)],
                    )

        zero_acc()
        plsc.subcore_barrier()
        stage(table_hbm.at[c], pairsa_hbm, mid_hbm, True)
        zero_acc()
        plsc.subcore_barrier()
        stage(mid_hbm.at[c], pairsb_hbm, out_hbm, False)

    return sc_lmm


def _sc_lmm_call(table, pairs_a, pairs_b, scale_pad):
    return _build_sc_lmm()(table, pairs_a, pairs_b, scale_pad)


def _mm1_body(x_ref, w_ref, b_ref, a_ref, out_ref):
    y = lax.dot_general(
        x_ref[...], w_ref[...], (((1,), (1,)), ((), ())),
        preferred_element_type=jnp.float32,
    )
    out_ref[...] = ((y + b_ref[...]) * a_ref[...])[None]


def _tc_mm1(X, W1, b1, acol):
    return pl.pallas_call(
        _mm1_body,
        grid=(NC, N // ROWS),
        in_specs=[
            pl.BlockSpec((ROWS, D), lambda c, i: (i, 0)),
            pl.BlockSpec((DH, D), lambda c, i: (c, 0)),
            pl.BlockSpec((1, DH), lambda c, i: (0, c)),
            pl.BlockSpec((ROWS, 1), lambda c, i: (i, 0)),
        ],
        out_specs=pl.BlockSpec((1, ROWS, DH), lambda c, i: (c, i, 0)),
        out_shape=jax.ShapeDtypeStruct((NC, SACC, DH), jnp.float32),
    )(X, W1, b1.reshape(1, D), acol)


def _mm2_body(z_ref, a_ref, w_ref, b_ref, out_ref):
    a = a_ref[...]
    h0 = jnp.maximum(z_ref[0] * a, 0.0)
    h1 = jnp.maximum(z_ref[1] * a, 0.0)
    p = lax.dot_general(
        h0, w_ref[:, :DH], (((1,), (1,)), ((), ())),
        preferred_element_type=jnp.float32,
    ) + lax.dot_general(
        h1, w_ref[:, DH:], (((1,), (1,)), ((), ())),
        preferred_element_type=jnp.float32,
    )
    out_ref[...] = ((p + b_ref[...]) * a)[None]


def _tc_mm2(Zv, W2, b2, acol):
    return pl.pallas_call(
        _mm2_body,
        grid=(NC, N // ROWS),
        in_specs=[
            pl.BlockSpec((NC, ROWS, DH), lambda c, i: (0, i, 0)),
            pl.BlockSpec((ROWS, 1), lambda c, i: (i, 0)),
            pl.BlockSpec((DH, D), lambda c, i: (c, 0)),
            pl.BlockSpec((1, DH), lambda c, i: (0, c)),
        ],
        out_specs=pl.BlockSpec((1, ROWS, DH), lambda c, i: (c, i, 0)),
        out_shape=jax.ShapeDtypeStruct((NC, SACC, DH), jnp.float32),
    )(Zv, acol, W2, b2.reshape(1, D))


def _final_body(z_ref, a_ref, out_ref):
    out_ref[...] = z_ref[0] * a_ref[...]


def _tc_final(Zv, acol):
    return pl.pallas_call(
        _final_body,
        grid=(NC, N // ROWS),
        in_specs=[
            pl.BlockSpec((1, ROWS, DH), lambda c, i: (c, i, 0)),
            pl.BlockSpec((ROWS, 1), lambda c, i: (i, 0)),
        ],
        out_specs=pl.BlockSpec((ROWS, DH), lambda c, i: (i, c)),
        out_shape=jax.ShapeDtypeStruct((N, D), jnp.float32),
    )(Zv, acol)


def _pack_pairs(gidx, sidx, gmod, spad_base):
    npad = NNZ_PAD - NNZ
    fill = jnp.arange(npad, dtype=jnp.int32)
    g = jnp.concatenate([gidx.astype(jnp.int32), fill % gmod])
    s = jnp.concatenate([sidx.astype(jnp.int32),
                         spad_base + fill % (SACC - spad_base)])
    g3 = g.reshape(NT, CH, 1, K)
    s3 = s.reshape(NT, CH, 1, K)
    return jnp.concatenate([g3, s3], axis=2).reshape(NT * CH, 2, K)


def kernel(X, W1, b1, W2, b2, node_idx, edge_idx):
    pairs_ne = _pack_pairs(node_idx, edge_idx, N, M)
    pairs_en = _pack_pairs(edge_idx, node_idx, M, N)

    dvp, dep = _sc_degrees_call(node_idx, edge_idx)
    a_row, einv_row = _tc_degrees(dvp, dep)
    acol = a_row.reshape(N, 1)
    einv_pad = jnp.concatenate(
        [einv_row.reshape(M), jnp.ones((SACC - M,), jnp.float32)])

    y1 = _tc_mm1(X, W1, b1, acol)
    _, zv = _sc_lmm_call(y1, pairs_ne, pairs_en, einv_pad)
    y2 = _tc_mm2(zv, W2, b2, acol)
    _, zv2 = _sc_lmm_call(y2, pairs_ne, pairs_en, einv_pad)
    return _tc_final(zv2, acol)


# speedup vs baseline: 1.1044x; 1.0221x over previous
import functools

import jax
import jax.numpy as jnp
from jax import lax
from jax.experimental import pallas as pl
from jax.experimental.pallas import tpu as pltpu
from jax.experimental.pallas import tpu_sc as plsc

N = 10000
M = 10000
NNZ = 160000
D = 256
DH = 128
NC = 2
NT = 16
K = 128
PT = 10240
CH = PT // K
PB = 20
NNZ_PAD = NT * PT
SACC = 10240
ZR = 32
ROWS = 10000
HPT = NNZ // NT


def _mesh():
    return plsc.VectorSubcoreMesh(core_axis_name="c", subcore_axis_name="s")


@functools.cache
def _build_sc_degrees():
    @functools.partial(
        pl.kernel,
        out_type=(
            jax.ShapeDtypeStruct((NT, N), jnp.float32),
            jax.ShapeDtypeStruct((NT, M), jnp.float32),
        ),
        mesh=_mesh(),
        compiler_params=pltpu.CompilerParams(needs_layout_passes=False),
        scratch_types=[
            pltpu.VMEM((HPT,), jnp.int32),
            pltpu.VMEM((N,), jnp.float32),
        ],
    )
    def sc_degrees(node_hbm, edge_hbm, dvp_hbm, dep_hbm, idx_v, hist_v):
        c = lax.axis_index("c")
        t = lax.axis_index("s")

        def do_hist(src_hbm, out_hbm):
            pltpu.sync_copy(src_hbm.at[pl.ds(t * HPT, HPT)], idx_v)

            def zero(i, carry):
                hist_v[pl.ds(i * 16, 16)] = jnp.zeros((16,), jnp.float32)
                return carry

            lax.fori_loop(0, N // 16, zero, 0)

            ones = jnp.ones((16,), jnp.float32)

            def acc(i, carry):
                idx = idx_v[pl.ds(i * 16, 16)]
                plsc.addupdate_scatter(hist_v, [idx], ones)
                return carry

            lax.fori_loop(0, HPT // 16, acc, 0)
            pltpu.sync_copy(hist_v, out_hbm.at[t])

        @pl.when(c == 0)
        def _():
            do_hist(node_hbm, dvp_hbm)

        @pl.when(c == 1)
        def _():
            do_hist(edge_hbm, dep_hbm)

    return sc_degrees


def _sc_degrees_call(node_idx, edge_idx):
    return _build_sc_degrees()(node_idx, edge_idx)


def _degrees_body(dvp_ref, dep_ref, a_ref, einv_ref):
    dv = jnp.sum(dvp_ref[...], axis=0, keepdims=True)
    de = jnp.sum(dep_ref[...], axis=0, keepdims=True)
    a_ref[...] = lax.rsqrt(dv)
    einv_ref[...] = 1.0 / de


def _tc_degrees(dvp, dep):
    return pl.pallas_call(
        _degrees_body,
        out_shape=[
            jax.ShapeDtypeStruct((1, N), jnp.float32),
            jax.ShapeDtypeStruct((1, M), jnp.float32),
        ],
    )(dvp, dep)


@functools.cache
def _build_sc_lmm():
    @functools.partial(
        pl.kernel,
        out_type=(
            jax.ShapeDtypeStruct((NC, SACC, DH), jnp.float32),
            jax.ShapeDtypeStruct((NC, SACC, DH), jnp.float32),
        ),
        mesh=_mesh(),
        compiler_params=pltpu.CompilerParams(needs_layout_passes=False),
        scratch_types=[
            pltpu.VMEM((2, PB, 2, K), jnp.int32),
            pltpu.VMEM((2, K, DH), jnp.float32),
            pltpu.VMEM((SACC // NT,), jnp.float32),
            pltpu.VMEM((ZR, DH), jnp.float32),
            pltpu.VMEM_SHARED((SACC, DH), jnp.float32),
            pltpu.SemaphoreType.DMA,
            pltpu.SemaphoreType.DMA,
        ],
    )
    def sc_lmm(table_hbm, pairsa_hbm, pairsb_hbm, scale_hbm,
               mid_hbm, out_hbm, pbuf, rows, ebuf, zbuf, acc, gsem0, gsem1):
        c = lax.axis_index("c")
        t = lax.axis_index("s")
        gsems = (gsem0, gsem1)
        rpt = SACC // NT
        base = t * rpt
        nblk = rpt // K

        def zset(i, carry):
            r = i // (DH // 16)
            col = (i % (DH // 16)) * 16
            zbuf[r, pl.ds(col, 16)] = jnp.zeros((16,), jnp.float32)
            return carry

        lax.fori_loop(0, ZR * (DH // 16), zset, 0)
        pltpu.sync_copy(scale_hbm.at[pl.ds(base, rpt)], ebuf)

        def zero_acc():
            def zfire(i, carry):
                pltpu.async_copy(zbuf, acc.at[pl.ds(base + i * ZR, ZR)], gsem0)
                return carry

            lax.fori_loop(0, rpt // ZR, zfire, 0)

            def zwait(i, carry):
                pltpu.make_async_copy(
                    zbuf, acc.at[pl.ds(base + i * ZR, ZR)], gsem0
                ).wait()
                return carry

            lax.fori_loop(0, rpt // ZR, zwait, 0)

        def stage(tbl, pairs_hbm, dst_hbm, scaled):
            def load_block(b):
                pltpu.sync_copy(
                    pairs_hbm.at[pl.ds(t * CH + b * PB, PB)],
                    pbuf.at[lax.rem(b, 2)],
                )

            def gidx(j):
                return pbuf.at[lax.rem(j // PB, 2), lax.rem(j, PB), 0]

            def sidx(j):
                return pbuf.at[lax.rem(j // PB, 2), lax.rem(j, PB), 1]

            def fire_gather(slot, j):
                pltpu.async_copy(tbl.at[gidx(j)], rows.at[slot], gsems[slot])

            def wait_gather(slot, j):
                pltpu.make_async_copy(
                    tbl.at[gidx(j)], rows.at[slot], gsems[slot]
                ).wait()

            def scatter_add(slot, j):
                pltpu.sync_copy(rows.at[slot], acc.at[sidx(j)], add=True)

            load_block(0)
            fire_gather(0, 0)

            def body(jj, carry):
                j0 = 2 * jj
                j1 = j0 + 1
                fire_gather(1, j1)
                wait_gather(0, j0)
                scatter_add(0, j0)

                @pl.when(jj != CH // 2 - 1)
                def _():
                    @pl.when(lax.rem(j0 + 2, PB) == 0)
                    def _():
                        load_block((j0 + 2) // PB)

                    fire_gather(0, j0 + 2)

                wait_gather(1, j1)
                scatter_add(1, j1)
                return carry

            lax.fori_loop(0, CH // 2, body, 0)
            plsc.subcore_barrier()

            if not scaled:
                pltpu.sync_copy(
                    acc.at[pl.ds(base, rpt)],
                    dst_hbm.at[c].at[pl.ds(base, rpt)],
                )
            else:
                pltpu.async_copy(acc.at[pl.ds(base, K)], rows.at[0], gsem0)
                dnums = lax.GatherDimensionNumbers(
                    offset_dims=(), collapsed_slice_dims=(0,),
                    start_index_map=(0,))
                for b in range(nblk):
                    sl = b % 2
                    pltpu.make_async_copy(
                        acc.at[pl.ds(base + b * K, K)], rows.at[sl], gsems[sl]
                    ).wait()
                    if b + 1 < nblk:
                        pltpu.async_copy(
                            acc.at[pl.ds(base + (b + 1) * K, K)],
                            rows.at[1 - sl],
                            gsems[1 - sl],
                        )

                    def scale_group(g, carry):
                        e16 = ebuf[pl.ds(b * K + g * 16, 16)]
                        for i in range(16):
                            r = g * 16 + i
                            e = lax.gather(
                                e16,
                                jnp.full((16, 1), i, jnp.int32),
                                dnums,
                                slice_sizes=(1,),
                                mode=lax.GatherScatterMode.PROMISE_IN_BOUNDS,
                            )
                            for q in range(DH // 16):
                                rows[sl, r, pl.ds(q * 16, 16)] = (
                                    rows[sl, r, pl.ds(q * 16, 16)] * e
                                )
                        return carry

                    lax.fori_loop(0, K // 16, scale_group, 0)
                    pltpu.sync_copy(
                        rows.at[sl],
                        dst_hbm.at[c].at[pl.ds(base + b * K, K)],
                    )

        zero_acc()
        plsc.subcore_barrier()
        stage(table_hbm.at[c], pairsa_hbm, mid_hbm, True)
        zero_acc()
        plsc.subcore_barrier()
        stage(mid_hbm.at[c], pairsb_hbm, out_hbm, False)

    return sc_lmm


def _sc_lmm_call(table, pairs_a, pairs_b, scale_pad):
    return _build_sc_lmm()(table, pairs_a, pairs_b, scale_pad)


def _mm1_body(x_ref, w_ref, b_ref, a_ref, out_ref):
    y = lax.dot_general(
        x_ref[...], w_ref[...], (((1,), (1,)), ((), ())),
        preferred_element_type=jnp.float32,
    )
    out_ref[...] = ((y + b_ref[...]) * a_ref[...])[None]


def _tc_mm1(X, W1, b1, acol):
    return pl.pallas_call(
        _mm1_body,
        grid=(NC, N // ROWS),
        in_specs=[
            pl.BlockSpec((ROWS, D), lambda c, i: (i, 0)),
            pl.BlockSpec((DH, D), lambda c, i: (c, 0)),
            pl.BlockSpec((1, DH), lambda c, i: (0, c)),
            pl.BlockSpec((ROWS, 1), lambda c, i: (i, 0)),
        ],
        out_specs=pl.BlockSpec((1, ROWS, DH), lambda c, i: (c, i, 0)),
        out_shape=jax.ShapeDtypeStruct((NC, SACC, DH), jnp.float32),
    )(X, W1, b1.reshape(1, D), acol)


def _mm2_body(z_ref, a_ref, w_ref, b_ref, out_ref):
    a = a_ref[...]
    h0 = jnp.maximum(z_ref[0] * a, 0.0)
    h1 = jnp.maximum(z_ref[1] * a, 0.0)
    p = lax.dot_general(
        h0, w_ref[:, :DH], (((1,), (1,)), ((), ())),
        preferred_element_type=jnp.float32,
    ) + lax.dot_general(
        h1, w_ref[:, DH:], (((1,), (1,)), ((), ())),
        preferred_element_type=jnp.float32,
    )
    out_ref[...] = ((p + b_ref[...]) * a)[None]


def _tc_mm2(Zv, W2, b2, acol):
    return pl.pallas_call(
        _mm2_body,
        grid=(NC, N // ROWS),
        in_specs=[
            pl.BlockSpec((NC, ROWS, DH), lambda c, i: (0, i, 0)),
            pl.BlockSpec((ROWS, 1), lambda c, i: (i, 0)),
            pl.BlockSpec((DH, D), lambda c, i: (c, 0)),
            pl.BlockSpec((1, DH), lambda c, i: (0, c)),
        ],
        out_specs=pl.BlockSpec((1, ROWS, DH), lambda c, i: (c, i, 0)),
        out_shape=jax.ShapeDtypeStruct((NC, SACC, DH), jnp.float32),
    )(Zv, acol, W2, b2.reshape(1, D))


def _final_body(z_ref, a_ref, out_ref):
    out_ref[...] = z_ref[0] * a_ref[...]


def _tc_final(Zv, acol):
    return pl.pallas_call(
        _final_body,
        grid=(NC, N // ROWS),
        in_specs=[
            pl.BlockSpec((1, ROWS, DH), lambda c, i: (c, i, 0)),
            pl.BlockSpec((ROWS, 1), lambda c, i: (i, 0)),
        ],
        out_specs=pl.BlockSpec((ROWS, DH), lambda c, i: (i, c)),
        out_shape=jax.ShapeDtypeStruct((N, D), jnp.float32),
    )(Zv, acol)


def _pack_pairs(gidx, sidx, gmod, spad_base):
    npad = NNZ_PAD - NNZ
    fill = jnp.arange(npad, dtype=jnp.int32)
    g = jnp.concatenate([gidx.astype(jnp.int32), fill % gmod])
    s = jnp.concatenate([sidx.astype(jnp.int32),
                         spad_base + fill % (SACC - spad_base)])
    g3 = g.reshape(NT, CH, 1, K)
    s3 = s.reshape(NT, CH, 1, K)
    return jnp.concatenate([g3, s3], axis=2).reshape(NT * CH, 2, K)


def kernel(X, W1, b1, W2, b2, node_idx, edge_idx):
    pairs_ne = _pack_pairs(node_idx, edge_idx, N, M)
    pairs_en = _pack_pairs(edge_idx, node_idx, M, N)

    dvp, dep = _sc_degrees_call(node_idx, edge_idx)
    a_row, einv_row = _tc_degrees(dvp, dep)
    acol = a_row.reshape(N, 1)
    einv_pad = jnp.concatenate(
        [einv_row.reshape(M), jnp.ones((SACC - M,), jnp.float32)])

    y1 = _tc_mm1(X, W1, b1, acol)
    _, zv = _sc_lmm_call(y1, pairs_ne, pairs_en, einv_pad)
    y2 = _tc_mm2(zv, W2, b2, acol)
    _, zv2 = _sc_lmm_call(y2, pairs_ne, pairs_en, einv_pad)
    return _tc_final(zv2, acol)
